# baseline (speedup 1.0000x reference)
import functools

import jax
import jax.numpy as jnp
from jax import lax
from jax.experimental import pallas as pl
from jax.experimental.pallas import tpu as pltpu
from jax.experimental.pallas import tpu_sc as plsc

NUM_FIELDS = 26
VOCAB = 100000
EMB_DIM = 50
BATCH = 16384

NC = 2
NS = 16
L = 16
NW = NC * NS

ROWS = BATCH * NUM_FIELDS
ROWS_PER_W = ROWS // NW
CHUNK = 128
NCHUNK = ROWS_PER_W // CHUNK
NG = CHUNK // L

W = 32
NWIN = 3
TOTAL_WORDS = NUM_FIELDS * VOCAB * EMB_DIM
VROWS = TOTAL_WORDS // W


def _make_gather():
  mesh = plsc.VectorSubcoreMesh(core_axis_name="c", subcore_axis_name="s")

  @functools.partial(
      pl.kernel,
      out_type=jax.ShapeDtypeStruct((ROWS, EMB_DIM), jnp.float32),
      mesh=mesh,
      scratch_types=[
          pltpu.VMEM((NCHUNK, CHUNK), jnp.int32),
          pltpu.VMEM((NWIN, CHUNK), jnp.int32),
          pltpu.VMEM((NWIN, CHUNK), jnp.int32),
          pltpu.VMEM((CHUNK,), jnp.int32),
          pltpu.VMEM((CHUNK,), jnp.int32),
          pltpu.VMEM((NWIN, CHUNK, W), jnp.float32),
          pltpu.VMEM((NWIN, CHUNK, W), jnp.float32),
          pltpu.VMEM((CHUNK, EMB_DIM), jnp.float32),
          pltpu.VMEM((CHUNK, EMB_DIM), jnp.float32),
          pltpu.SemaphoreType.DMA,
          pltpu.SemaphoreType.DMA,
          pltpu.SemaphoreType.DMA,
          pltpu.SemaphoreType.DMA,
      ],
      compiler_params=pltpu.CompilerParams(
          use_tc_tiling_on_sc=False, needs_layout_passes=False),
  )
  def grab(cat_hbm, table_hbm, out_hbm, idx_v, aq0, aq1, off0, off1,
           win0, win1, outb0, outb1, gs0, gs1, os0, os1):
    aq = (aq0, aq1)
    off = (off0, off1)
    win = (win0, win1)
    outb = (outb0, outb1)
    gsem = (gs0, gs1)
    osem = (os0, os1)

    wid = lax.axis_index("s") * NC + lax.axis_index("c")
    row_base = wid * ROWS_PER_W

    pltpu.sync_copy(cat_hbm.at[pl.ds(wid * NCHUNK, NCHUNK)], idx_v)

    lane = lax.iota(jnp.int32, L)

    def build(c, slot):
      for s in range(NG):
        col = s * L + lane
        cat_val = plsc.load_gather(idx_v, [jnp.full((L,), c, jnp.int32), col])
        r = row_base + c * CHUNK + col
        t = cat_val + (r % NUM_FIELDS) * VOCAB
        wrd = t * EMB_DIM
        a = lax.shift_right_logical(wrd, 5)
        off[slot][pl.ds(s * L, L)] = lax.bitwise_and(wrd, 31)
        for q in range(NWIN):
          aq[slot][q, pl.ds(s * L, L)] = a + q

    def fire_gathers(slot):
      for q in range(NWIN):
        pltpu.make_async_copy(
            table_hbm.at[aq[slot].at[q]], win[slot].at[q], gsem[slot]).start()

    def wait_gathers(slot):
      for q in range(NWIN):
        pltpu.make_async_copy(
            table_hbm.at[aq[slot].at[q]], win[slot].at[q], gsem[slot]).wait()

    def out_copy(c, slot):
      return pltpu.make_async_copy(
          outb[slot],
          out_hbm.at[pl.ds(row_base + c * CHUNK, CHUNK)],
          osem[slot])

    def realign(slot):
      for s in range(NG):
        jv = s * L + lane
        offv = off[slot][pl.ds(s * L, L)]
        for k in range(EMB_DIM):
          g = offv + k
          qv = lax.shift_right_logical(g, 5)
          cv = lax.bitwise_and(g, 31)
          val = plsc.load_gather(win[slot], [qv, jv, cv])
          plsc.store_scatter(
              outb[slot], [jv, jnp.full((L,), k, jnp.int32)], val)

    build(0, 0)
    fire_gathers(0)

    def step(c, slot):
      @pl.when(c + 1 < NCHUNK)
      def _():
        build(c + 1, 1 - slot)
        fire_gathers(1 - slot)

      wait_gathers(slot)

      @pl.when(c >= 2)
      def _():
        out_copy(c - 2, slot).wait()

      realign(slot)
      out_copy(c, slot).start()

    def loop_body(c2, carry):
      step(2 * c2, 0)
      step(2 * c2 + 1, 1)
      return carry

    lax.fori_loop(0, NCHUNK // 2, loop_body, 0)

    out_copy(NCHUNK - 2, 0).wait()
    out_copy(NCHUNK - 1, 1).wait()

  return grab


_gather = _make_gather()


def kernel(cat, tables):
  table32 = tables.reshape(VROWS, W)
  cat2d = cat.reshape(ROWS // CHUNK, CHUNK)
  out = _gather(cat2d, table32)
  return out.reshape(BATCH, NUM_FIELDS * EMB_DIM)
